# baseline (speedup 1.0000x reference)
import functools

import jax
import jax.numpy as jnp
from jax import lax
from jax.experimental import pallas as pl
from jax.experimental.pallas import tpu as pltpu
from jax.experimental.pallas import tpu_sc as plsc

D = 64
B = 4096
L = 200
TOTAL = B * L
CHUNK = 128
NROWS = TOTAL // CHUNK
NW = 32
ROWS_PER_W = NROWS // NW
NBUF = 4
K = 3

_mesh = plsc.VectorSubcoreMesh(core_axis_name="c", subcore_axis_name="s")


@functools.partial(
    pl.kernel,
    mesh=_mesh,
    out_type=(
        jax.ShapeDtypeStruct((NROWS, CHUNK, D), jnp.float32),
        jax.ShapeDtypeStruct((NROWS, CHUNK, D), jnp.float32),
    ),
    scratch_types=[
        pltpu.VMEM((ROWS_PER_W, CHUNK), jnp.int32),
        pltpu.VMEM((NBUF, CHUNK, D), jnp.float32),
    ]
    + [pltpu.SemaphoreType.DMA] * (2 * NBUF),
    compiler_params=pltpu.CompilerParams(use_tc_tiling_on_sc=False),
)
def _embed2(n2v, t2v, seq_i, ts_i, out_x, out_t, idx_v, rows, *sems):
    wid = lax.axis_index("s") * 2 + lax.axis_index("c")
    row0 = wid * ROWS_PER_W
    gs, os_ = sems[:NBUF], sems[NBUF:]
    for table, idx_hbm, out_hbm in ((n2v, seq_i, out_x), (t2v, ts_i, out_t)):
        pltpu.sync_copy(idx_hbm.at[pl.ds(row0, ROWS_PER_W)], idx_v)
        for b in range(K):
            pltpu.async_copy(table.at[idx_v.at[b]], rows.at[b], gs[b])

        def body(g, _, table=table, out_hbm=out_hbm):
            for b in range(NBUF):
                c = g * NBUF + b
                pltpu.make_async_copy(table.at[idx_v.at[c]], rows.at[b], gs[b]).wait()
                pltpu.async_copy(rows.at[b], out_hbm.at[row0 + c], os_[b])
                nb = (b + K) % NBUF
                def recycle(c=c, nb=nb, out_hbm=out_hbm):
                    pltpu.make_async_copy(
                        rows.at[nb], out_hbm.at[row0 + c - 1], os_[nb]
                    ).wait()
                if b == 0:
                    pl.when(g > 0)(recycle)
                else:
                    recycle()
                nxt = jnp.minimum(c + K, ROWS_PER_W - 1)
                pltpu.async_copy(table.at[idx_v.at[nxt]], rows.at[nb], gs[nb])
            return ()

        lax.fori_loop(0, ROWS_PER_W // NBUF, body, ())
        for b in range(K):
            pltpu.make_async_copy(
                table.at[idx_v.at[ROWS_PER_W - 1]], rows.at[b], gs[b]
            ).wait()
        pltpu.make_async_copy(
            rows.at[NBUF - 1], out_hbm.at[row0 + ROWS_PER_W - 1], os_[NBUF - 1]
        ).wait()


def kernel(seq, ts, node2vec, time2vec):
    seq_r = seq.reshape(NROWS, CHUNK).astype(jnp.int32)
    ts_r = ts.reshape(NROWS, CHUNK).astype(jnp.int32)
    x, t = _embed2(node2vec, time2vec, seq_r, ts_r)
    z = jnp.minimum(seq[0, 0], 0).astype(jnp.float32)
    return x.reshape(B, L, D) + z, t.reshape(B, L, D) + z


# speedup vs baseline: 1.4191x; 1.4191x over previous
import functools

import jax
import jax.numpy as jnp
from jax import lax
from jax.experimental import pallas as pl
from jax.experimental.pallas import tpu as pltpu
from jax.experimental.pallas import tpu_sc as plsc

D = 64
B = 4096
L = 200
TOTAL = B * L
CHUNK = 128
NROWS = TOTAL // CHUNK
NW = 32
ROWS_PER_W = NROWS // NW
NBUF = 8
K = 4
OLAG = NBUF - K

_mesh = plsc.VectorSubcoreMesh(core_axis_name="c", subcore_axis_name="s")


@functools.partial(
    pl.kernel,
    mesh=_mesh,
    out_type=(
        jax.ShapeDtypeStruct((NROWS, CHUNK, D), jnp.float32),
        jax.ShapeDtypeStruct((NROWS, CHUNK, D), jnp.float32),
    ),
    scratch_types=[
        pltpu.VMEM((ROWS_PER_W, CHUNK), jnp.int32),
        pltpu.VMEM((NBUF, CHUNK, D), jnp.float32),
    ]
    + [pltpu.SemaphoreType.DMA] * (2 * NBUF),
    compiler_params=pltpu.CompilerParams(use_tc_tiling_on_sc=False),
)
def _embed2(n2v, t2v, seq_i, ts_i, out_x, out_t, idx_v, rows, *sems):
    wid = lax.axis_index("s") * 2 + lax.axis_index("c")
    row0 = wid * ROWS_PER_W
    gs, os_ = sems[:NBUF], sems[NBUF:]
    for table, idx_hbm, out_hbm in ((n2v, seq_i, out_x), (t2v, ts_i, out_t)):
        pltpu.sync_copy(idx_hbm.at[pl.ds(row0, ROWS_PER_W)], idx_v)
        for b in range(K):
            pltpu.async_copy(table.at[idx_v.at[b]], rows.at[b], gs[b])

        def body(g, _, table=table, out_hbm=out_hbm):
            for b in range(NBUF):
                c = g * NBUF + b
                pltpu.make_async_copy(table.at[idx_v.at[c]], rows.at[b], gs[b]).wait()
                pltpu.async_copy(rows.at[b], out_hbm.at[row0 + c], os_[b])
                nb = (b + K) % NBUF
                def recycle(c=c, nb=nb, out_hbm=out_hbm):
                    pltpu.make_async_copy(
                        rows.at[nb], out_hbm.at[row0 + c - OLAG], os_[nb]
                    ).wait()
                if b < OLAG:
                    pl.when(g > 0)(recycle)
                else:
                    recycle()
                nxt = jnp.minimum(c + K, ROWS_PER_W - 1)
                pltpu.async_copy(table.at[idx_v.at[nxt]], rows.at[nb], gs[nb])
            return ()

        lax.fori_loop(0, ROWS_PER_W // NBUF, body, ())
        for b in range(K):
            pltpu.make_async_copy(
                table.at[idx_v.at[ROWS_PER_W - 1]], rows.at[b], gs[b]
            ).wait()
        for c in range(ROWS_PER_W - OLAG, ROWS_PER_W):
            pltpu.make_async_copy(
                rows.at[c % NBUF], out_hbm.at[row0 + c], os_[c % NBUF]
            ).wait()


def kernel(seq, ts, node2vec, time2vec):
    seq_r = seq.reshape(NROWS, CHUNK).astype(jnp.int32)
    ts_r = ts.reshape(NROWS, CHUNK).astype(jnp.int32)
    x, t = _embed2(node2vec, time2vec, seq_r, ts_r)
    return x.reshape(B, L, D), t.reshape(B, L, D)


# speedup vs baseline: 1.6692x; 1.1763x over previous
import functools

import jax
import jax.numpy as jnp
from jax import lax
from jax.experimental import pallas as pl
from jax.experimental.pallas import tpu as pltpu
from jax.experimental.pallas import tpu_sc as plsc

D = 64
W = 128
B = 4096
L = 200
TOTAL = B * L
CHUNK = 128
NROWS = TOTAL // CHUNK
NW = 32
ROWS_PER_W = NROWS // NW
PHASES = (104, 96)
NBUF = 4
OB = 2

_mesh = plsc.VectorSubcoreMesh(core_axis_name="c", subcore_axis_name="s")


@functools.partial(
    pl.kernel,
    mesh=_mesh,
    out_type=(
        jax.ShapeDtypeStruct((TOTAL, D), jnp.float32),
        jax.ShapeDtypeStruct((TOTAL, D), jnp.float32),
    ),
    scratch_types=[
        pltpu.VMEM((PHASES[0], CHUNK), jnp.int32),
        pltpu.VMEM((NBUF, CHUNK, W), jnp.float32),
        pltpu.VMEM((OB, CHUNK, D), jnp.float32),
    ]
    + [pltpu.SemaphoreType.DMA] * (NBUF + OB),
)
def _embed2(n2v, t2v, seq_i, ts_i, out_x, out_t, idx_v, awide, bpack, *sems):
    wid = lax.axis_index("s") * 2 + lax.axis_index("c")
    gs, os_ = sems[:NBUF], sems[NBUF:]
    for table, idx_hbm, out_hbm in ((n2v, seq_i, out_x), (t2v, ts_i, out_t)):
        for poff, plen in ((0, PHASES[0]), (PHASES[0], PHASES[1])):
            base = wid * ROWS_PER_W + poff
            pltpu.sync_copy(
                idx_hbm.at[pl.ds(base, plen)], idx_v.at[pl.ds(0, plen)]
            )
            for b in range(NBUF - 1):
                pltpu.async_copy(table.at[idx_v.at[b]], awide.at[b], gs[b])

            def body(g, _, table=table, out_hbm=out_hbm, base=base, plen=plen):
                for b in range(NBUF):
                    c = g * NBUF + b
                    o = b % OB
                    pltpu.make_async_copy(
                        table.at[idx_v.at[c]], awide.at[b], gs[b]
                    ).wait()
                    def recycle(c=c, o=o, out_hbm=out_hbm, base=base):
                        pltpu.make_async_copy(
                            bpack.at[o],
                            out_hbm.at[pl.ds((base + c - OB) * CHUNK, CHUNK)],
                            os_[o],
                        ).wait()
                    if b < OB:
                        pl.when(g > 0)(recycle)
                    else:
                        recycle()
                    def crow(r, _, b=b, o=o):
                        for j in range(D // 16):
                            bpack[o, r, pl.ds(j * 16, 16)] = awide[b, r, pl.ds(j * 16, 16)]
                        return ()
                    lax.fori_loop(0, CHUNK, crow, ())
                    nb = (b + NBUF - 1) % NBUF
                    nxt = jnp.minimum(c + NBUF - 1, plen - 1)
                    pltpu.async_copy(table.at[idx_v.at[nxt]], awide.at[nb], gs[nb])
                    pltpu.async_copy(
                        bpack.at[o],
                        out_hbm.at[pl.ds((base + c) * CHUNK, CHUNK)],
                        os_[o],
                    )
                return ()

            lax.fori_loop(0, plen // NBUF, body, ())
            for b in range(NBUF - 1):
                pltpu.make_async_copy(
                    table.at[idx_v.at[plen - 1]], awide.at[b], gs[b]
                ).wait()
            for c in range(plen - OB, plen):
                pltpu.make_async_copy(
                    bpack.at[c % OB],
                    out_hbm.at[pl.ds((base + c) * CHUNK, CHUNK)],
                    os_[c % OB],
                ).wait()


def kernel(seq, ts, node2vec, time2vec):
    seq_r = seq.reshape(NROWS, CHUNK).astype(jnp.int32)
    ts_r = ts.reshape(NROWS, CHUNK).astype(jnp.int32)
    n2v_p = jnp.pad(node2vec, ((0, 0), (0, W - D)))
    t2v_p = jnp.pad(time2vec, ((0, 0), (0, W - D)))
    x, t = _embed2(n2v_p, t2v_p, seq_r, ts_r)
    return x.reshape(B, L, D), t.reshape(B, L, D)
